# Initial kernel scaffold; baseline (speedup 1.0000x reference)
import jax
import jax.numpy as jnp
from jax.experimental import pallas as pl


def kernel(node_features, edge_index, batch, W1, b1, W2, b2, Wl, bl):
    raise NotImplementedError("write your pallas kernel here")


# speedup vs baseline: 28.4024x; 28.4024x over previous
import functools

import jax
import jax.numpy as jnp
from jax import lax
from jax.experimental import pallas as pl
from jax.experimental.pallas import tpu as pltpu
from jax.experimental.pallas import tpu_sc as plsc

N = 10000
D = 128
H = 128
C = 2
G = 64
E = 320000

PADR = 112
NP = N + PADR
NS = 16
NSC = 2
NW = NS * NSC
CHW = 128
NCH = 80
TPW = CHW * NCH
EPAD = TPW * NW
ROWS2D = EPAD // CHW
RPT = NP // NS

def _copy_idx_row(src2d, c, dst1d):
    for j in range(CHW // 16):
        dst1d[pl.ds(j * 16, 16)] = src2d[c, pl.ds(j * 16, 16)]


@functools.lru_cache(maxsize=None)
def _sc_mesh():
    return plsc.VectorSubcoreMesh(
        core_axis_name="c", subcore_axis_name="s",
        num_cores=NSC, num_subcores=NS)


@functools.lru_cache(maxsize=None)
def _deg_kernel_fn():
    return pl.kernel(
        _deg_body,
        out_type=jax.ShapeDtypeStruct((NSC, NP, H), jnp.float32),
        mesh=_sc_mesh(),
        scratch_types=[
            pltpu.VMEM((NCH, CHW), jnp.int32),
            pltpu.VMEM((CHW,), jnp.int32),
            pltpu.VMEM((CHW, H), jnp.float32),
            pltpu.VMEM((CHW, H), jnp.float32),
            pltpu.VMEM_SHARED((NP, H), jnp.float32),
        ],
    )


def _deg_body(dsts_hbm, out_hbm, dst_v, didx, zb_v, ones_v, deg_sh):
    cid = lax.axis_index("c")
    sid = lax.axis_index("s")
    wid = cid * NS + sid
    base = sid * RPT

    pltpu.sync_copy(dsts_hbm.at[pl.ds(wid * NCH, NCH)], dst_v)

    @pl.loop(0, CHW)
    def _fill(i):
        for j in range(H // 16):
            zb_v[i, pl.ds(j * 16, 16)] = jnp.zeros((16,), jnp.float32)
            ones_v[i, pl.ds(j * 16, 16)] = jnp.ones((16,), jnp.float32)

    for k in range(RPT // CHW):
        pltpu.sync_copy(zb_v, deg_sh.at[pl.ds(base + k * CHW, CHW)])
    rem = RPT % CHW
    pltpu.sync_copy(zb_v.at[pl.ds(0, rem)],
                    deg_sh.at[pl.ds(base + (RPT // CHW) * CHW, rem)])
    plsc.subcore_barrier()

    @pl.loop(0, NCH)
    def _scatter(c):
        _copy_idx_row(dst_v, c, didx)
        pltpu.sync_copy(ones_v, deg_sh.at[didx], add=True)

    plsc.subcore_barrier()
    pltpu.sync_copy(deg_sh.at[pl.ds(base, RPT)],
                    out_hbm.at[cid, pl.ds(base, RPT)])


@functools.lru_cache(maxsize=None)
def _agg_kernel_fn():
    return pl.kernel(
        _agg_body,
        out_type=jax.ShapeDtypeStruct((NSC, NP, H), jnp.float32),
        mesh=_sc_mesh(),
        scratch_types=[
            pltpu.VMEM((NCH // 2, CHW), jnp.int32),
            pltpu.VMEM((NCH // 2, CHW), jnp.int32),
            pltpu.VMEM((CHW,), jnp.int32),
            pltpu.VMEM((CHW,), jnp.int32),
            pltpu.VMEM((CHW,), jnp.int32),
            pltpu.VMEM((CHW,), jnp.int32),
            pltpu.VMEM((CHW, H), jnp.float32),
            pltpu.VMEM((CHW, H), jnp.float32),
            pltpu.VMEM_SHARED((NP, H), jnp.float32),
            pltpu.SemaphoreType.DMA,
            pltpu.SemaphoreType.DMA,
        ],
    )


def _agg_body(table_hbm, srcs_hbm, dsts_hbm, out_hbm,
              src_v, dst_v, sidx0, sidx1, didx0, didx1, rows0, rows1,
              agg_sh, sem0, sem1):
    cid = lax.axis_index("c")
    sid = lax.axis_index("s")
    wid = cid * NS + sid
    base = sid * RPT

    @pl.loop(0, CHW)
    def _fill(i):
        for j in range(H // 16):
            rows0[i, pl.ds(j * 16, 16)] = jnp.zeros((16,), jnp.float32)

    for k in range(RPT // CHW):
        pltpu.sync_copy(rows0, agg_sh.at[pl.ds(base + k * CHW, CHW)])
    rem = RPT % CHW
    pltpu.sync_copy(rows0.at[pl.ds(0, rem)],
                    agg_sh.at[pl.ds(base + (RPT // CHW) * CHW, rem)])
    plsc.subcore_barrier()

    HALF = NCH // 2
    for h in range(2):
        pltpu.sync_copy(srcs_hbm.at[pl.ds(wid * NCH + h * HALF, HALF)], src_v)
        pltpu.sync_copy(dsts_hbm.at[pl.ds(wid * NCH + h * HALF, HALF)], dst_v)

        _copy_idx_row(src_v, 0, sidx0)
        pltpu.async_copy(table_hbm.at[sidx0], rows0, sem0)

        @pl.loop(0, HALF // 2)
        def _run(k):
            cc = 2 * k
            _copy_idx_row(src_v, cc + 1, sidx1)
            pltpu.async_copy(table_hbm.at[sidx1], rows1, sem1)
            _copy_idx_row(dst_v, cc, didx0)
            pltpu.make_async_copy(table_hbm.at[sidx0], rows0, sem0).wait()
            pltpu.sync_copy(rows0, agg_sh.at[didx0], add=True)

            @pl.when(k < HALF // 2 - 1)
            def _next():
                _copy_idx_row(src_v, cc + 2, sidx0)
                pltpu.async_copy(table_hbm.at[sidx0], rows0, sem0)

            _copy_idx_row(dst_v, cc + 1, didx1)
            pltpu.make_async_copy(table_hbm.at[sidx1], rows1, sem1).wait()
            pltpu.sync_copy(rows1, agg_sh.at[didx1], add=True)

    plsc.subcore_barrier()
    pltpu.sync_copy(agg_sh.at[pl.ds(base, RPT)],
                    out_hbm.at[cid, pl.ds(base, RPT)])


def _dinv_from_degp(degp_ref):
    d = degp_ref[0][:, 0:1] + degp_ref[1][:, 0:1] + 1.0
    return lax.rsqrt(d)


def _tk1_body(nf_ref, w1_ref, degp_ref, xs_ref):
    dinv = _dinv_from_degp(degp_ref)
    x = jnp.dot(nf_ref[...], w1_ref[...], preferred_element_type=jnp.float32)
    xs_ref[...] = x * dinv


_tk1 = pl.pallas_call(
    _tk1_body, out_shape=jax.ShapeDtypeStruct((NP, H), jnp.float32))


def _tk2_body(aggp_ref, xs1_ref, degp_ref, b1_ref, w2_ref, xs2_ref):
    dinv = _dinv_from_degp(degp_ref)
    h1 = jnp.maximum(
        (aggp_ref[0] + aggp_ref[1] + xs1_ref[...]) * dinv + b1_ref[...], 0.0)
    xs2 = jnp.dot(h1, w2_ref[...], preferred_element_type=jnp.float32) * dinv
    row = lax.broadcasted_iota(jnp.int32, (NP, 1), 0)
    xs2_ref[...] = jnp.where(row < N, xs2, 0.0)


_tk2 = pl.pallas_call(
    _tk2_body, out_shape=jax.ShapeDtypeStruct((NP, H), jnp.float32))


def _tk3_body(aggp_ref, xs2_ref, degp_ref, b2_ref, batch_ref, wl_ref, bl_ref,
              out_ref):
    dinv = _dinv_from_degp(degp_ref)
    h2 = jnp.maximum(
        (aggp_ref[0] + aggp_ref[1] + xs2_ref[...]) * dinv + b2_ref[...], 0.0)
    bv = batch_ref[0:1, :]
    gids = lax.broadcasted_iota(jnp.int32, (G, 1), 0).astype(jnp.float32)
    mask = (bv == gids).astype(jnp.float32)
    sums = jnp.dot(mask, h2, preferred_element_type=jnp.float32)
    cnt = jnp.sum(mask, axis=1, keepdims=True)
    pooled = sums / jnp.maximum(cnt, 1.0)
    out_ref[...] = (
        jnp.dot(pooled, wl_ref[...], preferred_element_type=jnp.float32)
        + bl_ref[...])


_tk3 = pl.pallas_call(
    _tk3_body, out_shape=jax.ShapeDtypeStruct((G, C), jnp.float32))


def kernel(node_features, edge_index, batch, W1, b1, W2, b2, Wl, bl):
    src = edge_index[0]
    dst = edge_index[1]
    pad_idx = N + (jnp.arange(EPAD - E, dtype=jnp.int32) % PADR)
    srcs2d = jnp.concatenate([src, pad_idx]).reshape(ROWS2D, CHW)
    dsts2d = jnp.concatenate([dst, pad_idx]).reshape(ROWS2D, CHW)
    nf_pad = jnp.pad(node_features, ((0, PADR), (0, 0)))
    batch_b = jnp.broadcast_to(
        jnp.pad(batch, (0, PADR), constant_values=G).astype(jnp.float32)[None],
        (8, NP))
    b1_2d = b1[None, :]
    b2_2d = b2[None, :]
    bl_2d = bl[None, :]

    degp = _deg_kernel_fn()(dsts2d)
    xs1 = _tk1(nf_pad, W1, degp)
    aggp1 = _agg_kernel_fn()(xs1, srcs2d, dsts2d)
    xs2 = _tk2(aggp1, xs1, degp, b1_2d, W2)
    aggp2 = _agg_kernel_fn()(xs2, srcs2d, dsts2d)
    return _tk3(aggp2, xs2, degp, b2_2d, batch_b, Wl, bl_2d)


# speedup vs baseline: 28.5066x; 1.0037x over previous
import functools

import jax
import jax.numpy as jnp
from jax import lax
from jax.experimental import pallas as pl
from jax.experimental.pallas import tpu as pltpu
from jax.experimental.pallas import tpu_sc as plsc

N = 10000
D = 128
H = 128
C = 2
G = 64
E = 320000

PADR = 112
NP = N + PADR
NS = 16
NSC = 2
NW = NS * NSC
CHW = 128
NCH = 80
TPW = CHW * NCH
EPAD = TPW * NW
ROWS2D = EPAD // CHW
RPT = NP // NS

def _copy_idx_row(src2d, c, dst1d):
    for j in range(CHW // 16):
        dst1d[pl.ds(j * 16, 16)] = src2d[c, pl.ds(j * 16, 16)]


@functools.lru_cache(maxsize=None)
def _sc_mesh():
    return plsc.VectorSubcoreMesh(
        core_axis_name="c", subcore_axis_name="s",
        num_cores=NSC, num_subcores=NS)


@functools.lru_cache(maxsize=None)
def _deg_kernel_fn():
    return pl.kernel(
        _deg_body,
        out_type=jax.ShapeDtypeStruct((NSC, NP, H), jnp.float32),
        mesh=_sc_mesh(),
        scratch_types=[
            pltpu.VMEM((NCH, CHW), jnp.int32),
            pltpu.VMEM((CHW,), jnp.int32),
            pltpu.VMEM((CHW, H), jnp.float32),
            pltpu.VMEM((CHW, H), jnp.float32),
            pltpu.VMEM_SHARED((NP, H), jnp.float32),
        ],
    )


def _deg_body(dsts_hbm, out_hbm, dst_v, didx, zb_v, ones_v, deg_sh):
    cid = lax.axis_index("c")
    sid = lax.axis_index("s")
    wid = cid * NS + sid
    base = sid * RPT

    pltpu.sync_copy(dsts_hbm.at[pl.ds(wid * NCH, NCH)], dst_v)

    @pl.loop(0, CHW)
    def _fill(i):
        for j in range(H // 16):
            zb_v[i, pl.ds(j * 16, 16)] = jnp.zeros((16,), jnp.float32)
            ones_v[i, pl.ds(j * 16, 16)] = jnp.ones((16,), jnp.float32)

    for k in range(RPT // CHW):
        pltpu.sync_copy(zb_v, deg_sh.at[pl.ds(base + k * CHW, CHW)])
    rem = RPT % CHW
    pltpu.sync_copy(zb_v.at[pl.ds(0, rem)],
                    deg_sh.at[pl.ds(base + (RPT // CHW) * CHW, rem)])
    plsc.subcore_barrier()

    @pl.loop(0, NCH)
    def _scatter(c):
        _copy_idx_row(dst_v, c, didx)
        pltpu.sync_copy(ones_v, deg_sh.at[didx], add=True)

    plsc.subcore_barrier()
    pltpu.sync_copy(deg_sh.at[pl.ds(base, RPT)],
                    out_hbm.at[cid, pl.ds(base, RPT)])


@functools.lru_cache(maxsize=None)
def _agg_kernel_fn():
    return pl.kernel(
        _agg_body,
        out_type=jax.ShapeDtypeStruct((NSC, NP, H), jnp.float32),
        mesh=_sc_mesh(),
        scratch_types=[
            pltpu.VMEM((NCH // 2, CHW), jnp.int32),
            pltpu.VMEM((NCH // 2, CHW), jnp.int32),
            pltpu.VMEM((CHW,), jnp.int32),
            pltpu.VMEM((CHW,), jnp.int32),
            pltpu.VMEM((CHW,), jnp.int32),
            pltpu.VMEM((CHW,), jnp.int32),
            pltpu.VMEM((CHW, H), jnp.float32),
            pltpu.VMEM((CHW, H), jnp.float32),
            pltpu.VMEM_SHARED((NP, H), jnp.float32),
            pltpu.SemaphoreType.DMA,
            pltpu.SemaphoreType.DMA,
        ],
    )


def _agg_body(table_hbm, srcs_hbm, dsts_hbm, out_hbm,
              src_v, dst_v, sidx0, sidx1, didx0, didx1, rows0, rows1,
              agg_sh, sem0, sem1):
    cid = lax.axis_index("c")
    sid = lax.axis_index("s")
    wid = cid * NS + sid
    base = sid * RPT

    @pl.loop(0, CHW)
    def _fill(i):
        for j in range(H // 16):
            rows0[i, pl.ds(j * 16, 16)] = jnp.zeros((16,), jnp.float32)

    for k in range(RPT // CHW):
        pltpu.sync_copy(rows0, agg_sh.at[pl.ds(base + k * CHW, CHW)])
    rem = RPT % CHW
    pltpu.sync_copy(rows0.at[pl.ds(0, rem)],
                    agg_sh.at[pl.ds(base + (RPT // CHW) * CHW, rem)])
    plsc.subcore_barrier()

    HALF = NCH // 2
    for h in range(2):
        pltpu.sync_copy(srcs_hbm.at[pl.ds(wid * NCH + h * HALF, HALF)], src_v)
        pltpu.sync_copy(dsts_hbm.at[pl.ds(wid * NCH + h * HALF, HALF)], dst_v)

        _copy_idx_row(src_v, 0, sidx0)
        pltpu.async_copy(table_hbm.at[sidx0], rows0, sem0)

        @pl.loop(0, HALF // 2)
        def _run(k):
            cc = 2 * k
            _copy_idx_row(src_v, cc + 1, sidx1)
            pltpu.async_copy(table_hbm.at[sidx1], rows1, sem1)
            _copy_idx_row(dst_v, cc, didx0)
            pltpu.make_async_copy(table_hbm.at[sidx0], rows0, sem0).wait()
            pltpu.sync_copy(rows0, agg_sh.at[didx0], add=True)

            @pl.when(k < HALF // 2 - 1)
            def _next():
                _copy_idx_row(src_v, cc + 2, sidx0)
                pltpu.async_copy(table_hbm.at[sidx0], rows0, sem0)

            _copy_idx_row(dst_v, cc + 1, didx1)
            pltpu.make_async_copy(table_hbm.at[sidx1], rows1, sem1).wait()
            pltpu.sync_copy(rows1, agg_sh.at[didx1], add=True)

    plsc.subcore_barrier()
    pltpu.sync_copy(agg_sh.at[pl.ds(base, RPT)],
                    out_hbm.at[cid, pl.ds(base, RPT)])


def _tk1_body(nf_ref, w1_ref, degp_ref, xs_ref, dinv_ref):
    d = degp_ref[0][:, 0:1] + degp_ref[1][:, 0:1] + 1.0
    dinv = lax.rsqrt(d)
    dinv_ref[...] = dinv
    x = jnp.dot(nf_ref[...], w1_ref[...], preferred_element_type=jnp.float32)
    xs_ref[...] = x * dinv


_tk1 = pl.pallas_call(
    _tk1_body,
    out_shape=(jax.ShapeDtypeStruct((NP, H), jnp.float32),
               jax.ShapeDtypeStruct((NP, 1), jnp.float32)))


def _tk2_body(aggp_ref, xs1_ref, dinv_ref, b1_ref, w2_ref, xs2_ref):
    dinv = dinv_ref[...]
    h1 = jnp.maximum(
        (aggp_ref[0] + aggp_ref[1] + xs1_ref[...]) * dinv + b1_ref[...], 0.0)
    xs2 = jnp.dot(h1, w2_ref[...], preferred_element_type=jnp.float32) * dinv
    row = lax.broadcasted_iota(jnp.int32, (NP, 1), 0)
    xs2_ref[...] = jnp.where(row < N, xs2, 0.0)


_tk2 = pl.pallas_call(
    _tk2_body, out_shape=jax.ShapeDtypeStruct((NP, H), jnp.float32))


def _tk3_body(aggp_ref, xs2_ref, dinv_ref, b2_ref, batch_ref, wl_ref, bl_ref,
              out_ref):
    dinv = dinv_ref[...]
    h2 = jnp.maximum(
        (aggp_ref[0] + aggp_ref[1] + xs2_ref[...]) * dinv + b2_ref[...], 0.0)
    bv = batch_ref[0:1, :]
    gids = lax.broadcasted_iota(jnp.int32, (G, 1), 0).astype(jnp.float32)
    mask = (bv == gids).astype(jnp.float32)
    sums = jnp.dot(mask, h2, preferred_element_type=jnp.float32)
    cnt = jnp.sum(mask, axis=1, keepdims=True)
    pooled = sums / jnp.maximum(cnt, 1.0)
    out_ref[...] = (
        jnp.dot(pooled, wl_ref[...], preferred_element_type=jnp.float32)
        + bl_ref[...])


_tk3 = pl.pallas_call(
    _tk3_body, out_shape=jax.ShapeDtypeStruct((G, C), jnp.float32))


def kernel(node_features, edge_index, batch, W1, b1, W2, b2, Wl, bl):
    src = edge_index[0]
    dst = edge_index[1]
    pad_idx = N + (jnp.arange(EPAD - E, dtype=jnp.int32) % PADR)
    srcs2d = jnp.concatenate([src, pad_idx]).reshape(ROWS2D, CHW)
    dsts2d = jnp.concatenate([dst, pad_idx]).reshape(ROWS2D, CHW)
    nf_pad = jnp.pad(node_features, ((0, PADR), (0, 0)))
    batch_b = jnp.broadcast_to(
        jnp.pad(batch, (0, PADR), constant_values=G).astype(jnp.float32)[None],
        (8, NP))
    b1_2d = b1[None, :]
    b2_2d = b2[None, :]
    bl_2d = bl[None, :]

    degp = _deg_kernel_fn()(dsts2d)
    xs1, dinv = _tk1(nf_pad, W1, degp)
    aggp1 = _agg_kernel_fn()(xs1, srcs2d, dsts2d)
    xs2 = _tk2(aggp1, xs1, dinv, b1_2d, W2)
    aggp2 = _agg_kernel_fn()(xs2, srcs2d, dsts2d)
    return _tk3(aggp2, xs2, dinv, b2_2d, batch_b, Wl, bl_2d)
